# initial kernel scaffold (unmeasured)
import jax
import jax.numpy as jnp
from jax import lax
from jax.experimental import pallas as pl
from jax.experimental.pallas import tpu as pltpu

N_DEV = 4


def kernel(A, B):
    m_per, k = A.shape
    k2, n = B.shape
    assert k == k2
    half = m_per // 2

    def body(a_ref, b_ref, out_ref, comm_ref, outbuf, send_sems, recv_sems,
             copy_sems):
        my_pos = lax.axis_index("i")
        left = lax.rem(my_pos - 1 + N_DEV, N_DEV)
        right = lax.rem(my_pos + 1, N_DEV)

        barrier_sem = pltpu.get_barrier_semaphore()
        for nbr in (left, right):
            pl.semaphore_signal(
                barrier_sem, inc=1,
                device_id=(nbr,), device_id_type=pl.DeviceIdType.MESH,
            )
        pl.semaphore_wait(barrier_sem, 2)

        rdma0 = pltpu.make_async_remote_copy(
            src_ref=a_ref,
            dst_ref=comm_ref.at[0],
            send_sem=send_sems.at[0],
            recv_sem=recv_sems.at[0],
            device_id=(right,),
            device_id_type=pl.DeviceIdType.MESH,
        )
        rdma0.start()

        def compute_block(src, origin, slot0):
            for j in range(2):
                slot = (slot0 + j) % 2
                pltpu.make_async_copy(
                    outbuf.at[slot], outbuf.at[slot], copy_sems.at[slot]
                ).wait()
                outbuf[slot] = jnp.dot(
                    src[pl.ds(j * half, half), :], b_ref[:, :],
                    preferred_element_type=jnp.float32,
                )
                cp = pltpu.make_async_copy(
                    outbuf.at[slot],
                    out_ref.at[pl.ds(origin * m_per + j * half, half), :],
                    copy_sems.at[slot],
                )
                cp.start()

        for s in range(2):
            pl.semaphore_signal(copy_sems.at[s], inc=1)

        compute_block(a_ref, my_pos, 0)

        for h in range(N_DEV - 1):
            recv = pltpu.make_async_remote_copy(
                src_ref=comm_ref.at[h],
                dst_ref=comm_ref.at[h],
                send_sem=send_sems.at[h],
                recv_sem=recv_sems.at[h],
                device_id=(left,),
                device_id_type=pl.DeviceIdType.MESH,
            )
            recv.wait_recv()
            if h < N_DEV - 2:
                fwd = pltpu.make_async_remote_copy(
                    src_ref=comm_ref.at[h],
                    dst_ref=comm_ref.at[h + 1],
                    send_sem=send_sems.at[h + 1],
                    recv_sem=recv_sems.at[h + 1],
                    device_id=(right,),
                    device_id_type=pl.DeviceIdType.MESH,
                )
                fwd.start()
            origin = lax.rem(my_pos - 1 - h + N_DEV, N_DEV)
            compute_block(comm_ref[h], origin, 0)

        for h in range(N_DEV - 1):
            pltpu.make_async_remote_copy(
                src_ref=comm_ref.at[h] if h > 0 else a_ref,
                dst_ref=comm_ref.at[h],
                send_sem=send_sems.at[h],
                recv_sem=recv_sems.at[h],
                device_id=(right,),
                device_id_type=pl.DeviceIdType.MESH,
            ).wait_send()
        for s in range(2):
            pltpu.make_async_copy(
                outbuf.at[s], outbuf.at[s], copy_sems.at[s]
            ).wait()

    out_shape = jax.ShapeDtypeStruct((N_DEV * m_per, n), jnp.float32)
    return pl.pallas_call(
        body,
        out_shape=out_shape,
        in_specs=[
            pl.BlockSpec(memory_space=pltpu.VMEM),
            pl.BlockSpec(memory_space=pltpu.VMEM),
        ],
        out_specs=pl.BlockSpec(memory_space=pltpu.ANY),
        scratch_shapes=[
            pltpu.VMEM((N_DEV - 1, m_per, k), jnp.float32),
            pltpu.VMEM((2, half, n), jnp.float32),
            pltpu.SemaphoreType.DMA((N_DEV - 1,)),
            pltpu.SemaphoreType.DMA((N_DEV - 1,)),
            pltpu.SemaphoreType.DMA((2,)),
        ],
        compiler_params=pltpu.CompilerParams(collective_id=0),
    )(A, B)


# baseline (device time: 340502 ns/iter reference)
import jax
import jax.numpy as jnp
from jax import lax
from jax.experimental import pallas as pl
from jax.experimental.pallas import tpu as pltpu

N_DEV = 4


def kernel(A, B):
    m_per, k = A.shape
    k2, n = B.shape
    assert k == k2
    half = m_per // 2

    def body(a_ref, b_ref, out_ref, comm_ref, outbuf, send_sems, recv_sems,
             copy_sems):
        my_pos = lax.axis_index("i")
        left = lax.rem(my_pos - 1 + N_DEV, N_DEV)
        right = lax.rem(my_pos + 1, N_DEV)

        barrier_sem = pltpu.get_barrier_semaphore()
        for nbr in (left, right):
            pl.semaphore_signal(
                barrier_sem, inc=1,
                device_id=(nbr,), device_id_type=pl.DeviceIdType.MESH,
            )
        pl.semaphore_wait(barrier_sem, 2)

        rdma0 = pltpu.make_async_remote_copy(
            src_ref=a_ref,
            dst_ref=comm_ref.at[0],
            send_sem=send_sems.at[0],
            recv_sem=recv_sems.at[0],
            device_id=(right,),
            device_id_type=pl.DeviceIdType.MESH,
        )
        rdma0.start()

        def compute_block(src, origin, first=False):
            for j in range(2):
                slot = j
                if not first:
                    pltpu.make_async_copy(
                        outbuf.at[slot],
                        out_ref.at[pl.ds(origin * m_per + j * half, half), :],
                        copy_sems.at[slot],
                    ).wait()
                outbuf[slot] = jnp.dot(
                    src[pl.ds(j * half, half), :], b_ref[:, :],
                    preferred_element_type=jnp.float32,
                )
                cp = pltpu.make_async_copy(
                    outbuf.at[slot],
                    out_ref.at[pl.ds(origin * m_per + j * half, half), :],
                    copy_sems.at[slot],
                )
                cp.start()

        compute_block(a_ref, my_pos, first=True)

        for h in range(N_DEV - 1):
            recv = pltpu.make_async_remote_copy(
                src_ref=comm_ref.at[h],
                dst_ref=comm_ref.at[h],
                send_sem=send_sems.at[h],
                recv_sem=recv_sems.at[h],
                device_id=(left,),
                device_id_type=pl.DeviceIdType.MESH,
            )
            recv.wait_recv()
            if h < N_DEV - 2:
                fwd = pltpu.make_async_remote_copy(
                    src_ref=comm_ref.at[h],
                    dst_ref=comm_ref.at[h + 1],
                    send_sem=send_sems.at[h + 1],
                    recv_sem=recv_sems.at[h + 1],
                    device_id=(right,),
                    device_id_type=pl.DeviceIdType.MESH,
                )
                fwd.start()
            origin = lax.rem(my_pos - 1 - h + N_DEV, N_DEV)
            compute_block(comm_ref.at[h], origin)

        for h in range(N_DEV - 1):
            pltpu.make_async_remote_copy(
                src_ref=comm_ref.at[h] if h > 0 else a_ref,
                dst_ref=comm_ref.at[h],
                send_sem=send_sems.at[h],
                recv_sem=recv_sems.at[h],
                device_id=(right,),
                device_id_type=pl.DeviceIdType.MESH,
            ).wait_send()
        for s in range(2):
            pltpu.make_async_copy(
                outbuf.at[s], outbuf.at[s], copy_sems.at[s]
            ).wait()

    out_shape = jax.ShapeDtypeStruct((N_DEV * m_per, n), jnp.float32)
    return pl.pallas_call(
        body,
        out_shape=out_shape,
        in_specs=[
            pl.BlockSpec(memory_space=pltpu.VMEM),
            pl.BlockSpec(memory_space=pltpu.VMEM),
        ],
        out_specs=pl.BlockSpec(memory_space=pl.ANY),
        scratch_shapes=[
            pltpu.VMEM((N_DEV - 1, m_per, k), jnp.float32),
            pltpu.VMEM((2, half, n), jnp.float32),
            pltpu.SemaphoreType.DMA((N_DEV - 1,)),
            pltpu.SemaphoreType.DMA((N_DEV - 1,)),
            pltpu.SemaphoreType.DMA((2,)),
        ],
        compiler_params=pltpu.CompilerParams(
            collective_id=0, vmem_limit_bytes=100 * 1024 * 1024
        ),
    )(A, B)


# device time: 279305 ns/iter; 1.2191x vs baseline; 1.2191x over previous
import jax
import jax.numpy as jnp
from jax import lax
from jax.experimental import pallas as pl
from jax.experimental.pallas import tpu as pltpu

N_DEV = 4


def kernel(A, B):
    m_per, k = A.shape
    k2, n = B.shape
    assert k == k2
    half = m_per // 2
    q = m_per // 4

    def body(a_ref, b_ref, out_ref, c_own, c_top, c_bot, a_top_recv,
             a_bot_recv, a_send_sems, a_recv_sems, c_send_sems, c_recv_sems,
             copy_sems):
        my = lax.axis_index("i")
        left = (my + N_DEV - 1) % N_DEV
        right = (my + 1) % N_DEV
        diag = (my + 2) % N_DEV

        barrier_sem = pltpu.get_barrier_semaphore()
        for nbr in (left, right):
            pl.semaphore_signal(
                barrier_sem, inc=1,
                device_id=(nbr,), device_id_type=pl.DeviceIdType.MESH,
            )
        pl.semaphore_wait(barrier_sem, 2)

        a_top_send = pltpu.make_async_remote_copy(
            src_ref=a_ref.at[pl.ds(0, half), :],
            dst_ref=a_top_recv,
            send_sem=a_send_sems.at[0],
            recv_sem=a_recv_sems.at[0],
            device_id=(left,),
            device_id_type=pl.DeviceIdType.MESH,
        )
        a_top_send.start()
        a_bot_send = pltpu.make_async_remote_copy(
            src_ref=a_ref.at[pl.ds(half, half), :],
            dst_ref=a_bot_recv,
            send_sem=a_send_sems.at[1],
            recv_sem=a_recv_sems.at[1],
            device_id=(right,),
            device_id_type=pl.DeviceIdType.MESH,
        )
        a_bot_send.start()

        local_copies = []

        def ship(buf_at, grow, slot, peer):
            cp = pltpu.make_async_copy(
                buf_at, out_ref.at[pl.ds(grow, q), :], copy_sems.at[slot]
            )
            cp.start()
            local_copies.append(cp)
            pltpu.make_async_remote_copy(
                src_ref=buf_at,
                dst_ref=out_ref.at[pl.ds(grow, q), :],
                send_sem=c_send_sems.at[slot],
                recv_sem=c_recv_sems.at[slot],
                device_id=(peer,),
                device_id_type=pl.DeviceIdType.MESH,
            ).start()

        for qi, peer, slot in ((0, right, 0), (2, left, 2),
                               (1, right, 1), (3, left, 3)):
            c_own[pl.ds(qi * q, q), :] = jnp.dot(
                a_ref[pl.ds(qi * q, q), :], b_ref[:, :],
                preferred_element_type=jnp.float32,
            )
            ship(c_own.at[pl.ds(qi * q, q), :], my * m_per + qi * q,
                 slot, peer)

        a_top_send.wait_recv()
        for qi in range(2):
            c_top[pl.ds(qi * q, q), :] = jnp.dot(
                a_top_recv[pl.ds(qi * q, q), :], b_ref[:, :],
                preferred_element_type=jnp.float32,
            )
            ship(c_top.at[pl.ds(qi * q, q), :], right * m_per + qi * q,
                 4 + qi, left)

        a_bot_send.wait_recv()
        for qi in range(2):
            c_bot[pl.ds(qi * q, q), :] = jnp.dot(
                a_bot_recv[pl.ds(qi * q, q), :], b_ref[:, :],
                preferred_element_type=jnp.float32,
            )
            ship(c_bot.at[pl.ds(qi * q, q), :],
                 left * m_per + half + qi * q, 6 + qi, right)

        a_top_send.wait_send()
        a_bot_send.wait_send()
        for slot in range(8):
            pltpu.make_async_remote_copy(
                src_ref=c_own.at[pl.ds(0, q), :],
                dst_ref=out_ref.at[pl.ds(0, q), :],
                send_sem=c_send_sems.at[slot],
                recv_sem=c_recv_sems.at[slot],
                device_id=(right,),
                device_id_type=pl.DeviceIdType.MESH,
            ).wait_send()
        for cp in local_copies:
            cp.wait()
        inbound = (
            (0, left * m_per + 0 * q),
            (1, left * m_per + 1 * q),
            (2, right * m_per + 2 * q),
            (3, right * m_per + 3 * q),
            (4, diag * m_per + 0 * q),
            (5, diag * m_per + 1 * q),
            (6, diag * m_per + half + 0 * q),
            (7, diag * m_per + half + 1 * q),
        )
        for slot, grow in inbound:
            pltpu.make_async_remote_copy(
                src_ref=c_own.at[pl.ds(0, q), :],
                dst_ref=out_ref.at[pl.ds(grow, q), :],
                send_sem=c_send_sems.at[slot],
                recv_sem=c_recv_sems.at[slot],
                device_id=(left,),
                device_id_type=pl.DeviceIdType.MESH,
            ).wait_recv()

    out_shape = jax.ShapeDtypeStruct((N_DEV * m_per, n), jnp.float32)
    return pl.pallas_call(
        body,
        out_shape=out_shape,
        in_specs=[
            pl.BlockSpec(memory_space=pltpu.VMEM),
            pl.BlockSpec(memory_space=pltpu.VMEM),
        ],
        out_specs=pl.BlockSpec(memory_space=pl.ANY),
        scratch_shapes=[
            pltpu.VMEM((m_per, n), jnp.float32),
            pltpu.VMEM((half, n), jnp.float32),
            pltpu.VMEM((half, n), jnp.float32),
            pltpu.VMEM((half, k), jnp.float32),
            pltpu.VMEM((half, k), jnp.float32),
            pltpu.SemaphoreType.DMA((2,)),
            pltpu.SemaphoreType.DMA((2,)),
            pltpu.SemaphoreType.DMA((8,)),
            pltpu.SemaphoreType.DMA((8,)),
            pltpu.SemaphoreType.DMA((8,)),
        ],
        compiler_params=pltpu.CompilerParams(
            collective_id=0, vmem_limit_bytes=100 * 1024 * 1024
        ),
    )(A, B)


# device time: 279241 ns/iter; 1.2194x vs baseline; 1.0002x over previous
import jax
import jax.numpy as jnp
from jax import lax
from jax.experimental import pallas as pl
from jax.experimental.pallas import tpu as pltpu

N_DEV = 4


def kernel(A, B):
    m_per, k = A.shape
    k2, n = B.shape
    assert k == k2
    half = m_per // 2
    q = m_per // 4

    def body(a_ref, b_ref, out_ref, c_own, c_top, c_bot, a_top_recv,
             a_bot_recv, a_send_sems, a_recv_sems, c_send_sems, c_recv_sems,
             copy_sems):
        my = lax.axis_index("i")
        left = (my + N_DEV - 1) % N_DEV
        right = (my + 1) % N_DEV
        diag = (my + 2) % N_DEV

        barrier_sem = pltpu.get_barrier_semaphore()
        for nbr in (left, right):
            pl.semaphore_signal(
                barrier_sem, inc=1,
                device_id=(nbr,), device_id_type=pl.DeviceIdType.MESH,
            )
        pl.semaphore_wait(barrier_sem, 2)

        a_top_send = pltpu.make_async_remote_copy(
            src_ref=a_ref.at[pl.ds(0, half), :],
            dst_ref=a_top_recv,
            send_sem=a_send_sems.at[0],
            recv_sem=a_recv_sems.at[0],
            device_id=(left,),
            device_id_type=pl.DeviceIdType.MESH,
        )
        a_top_send.start()
        a_bot_send = pltpu.make_async_remote_copy(
            src_ref=a_ref.at[pl.ds(half, half), :],
            dst_ref=a_bot_recv,
            send_sem=a_send_sems.at[1],
            recv_sem=a_recv_sems.at[1],
            device_id=(right,),
            device_id_type=pl.DeviceIdType.MESH,
        )
        a_bot_send.start()

        local_copies = []

        def ship(buf_at, grow, slot, peer):
            cp = pltpu.make_async_copy(
                buf_at, out_ref.at[pl.ds(grow, q), :], copy_sems.at[slot]
            )
            cp.start()
            local_copies.append(cp)
            pltpu.make_async_remote_copy(
                src_ref=buf_at,
                dst_ref=out_ref.at[pl.ds(grow, q), :],
                send_sem=c_send_sems.at[slot],
                recv_sem=c_recv_sems.at[slot],
                device_id=(peer,),
                device_id_type=pl.DeviceIdType.MESH,
            ).start()

        for qi, peer, slot in ((0, right, 0), (2, left, 2),
                               (1, right, 1), (3, left, 3)):
            c_own[pl.ds(qi * q, q), :] = jnp.dot(
                a_ref[pl.ds(qi * q, q), :], b_ref[:, :],
                preferred_element_type=jnp.float32,
            )
            ship(c_own.at[pl.ds(qi * q, q), :], my * m_per + qi * q,
                 slot, peer)

        a_top_send.wait_recv()
        a_bot_send.wait_recv()
        for qi in range(2):
            c_top[pl.ds(qi * q, q), :] = jnp.dot(
                a_top_recv[pl.ds(qi * q, q), :], b_ref[:, :],
                preferred_element_type=jnp.float32,
            )
            ship(c_top.at[pl.ds(qi * q, q), :], right * m_per + qi * q,
                 4 + qi, left)
            c_bot[pl.ds(qi * q, q), :] = jnp.dot(
                a_bot_recv[pl.ds(qi * q, q), :], b_ref[:, :],
                preferred_element_type=jnp.float32,
            )
            ship(c_bot.at[pl.ds(qi * q, q), :],
                 left * m_per + half + qi * q, 6 + qi, right)

        a_top_send.wait_send()
        a_bot_send.wait_send()
        for slot in range(8):
            pltpu.make_async_remote_copy(
                src_ref=c_own.at[pl.ds(0, q), :],
                dst_ref=out_ref.at[pl.ds(0, q), :],
                send_sem=c_send_sems.at[slot],
                recv_sem=c_recv_sems.at[slot],
                device_id=(right,),
                device_id_type=pl.DeviceIdType.MESH,
            ).wait_send()
        for cp in local_copies:
            cp.wait()
        inbound = (
            (0, left * m_per + 0 * q),
            (1, left * m_per + 1 * q),
            (2, right * m_per + 2 * q),
            (3, right * m_per + 3 * q),
            (4, diag * m_per + 0 * q),
            (5, diag * m_per + 1 * q),
            (6, diag * m_per + half + 0 * q),
            (7, diag * m_per + half + 1 * q),
        )
        for slot, grow in inbound:
            pltpu.make_async_remote_copy(
                src_ref=c_own.at[pl.ds(0, q), :],
                dst_ref=out_ref.at[pl.ds(grow, q), :],
                send_sem=c_send_sems.at[slot],
                recv_sem=c_recv_sems.at[slot],
                device_id=(left,),
                device_id_type=pl.DeviceIdType.MESH,
            ).wait_recv()

    out_shape = jax.ShapeDtypeStruct((N_DEV * m_per, n), jnp.float32)
    return pl.pallas_call(
        body,
        out_shape=out_shape,
        in_specs=[
            pl.BlockSpec(memory_space=pltpu.VMEM),
            pl.BlockSpec(memory_space=pltpu.VMEM),
        ],
        out_specs=pl.BlockSpec(memory_space=pl.ANY),
        scratch_shapes=[
            pltpu.VMEM((m_per, n), jnp.float32),
            pltpu.VMEM((half, n), jnp.float32),
            pltpu.VMEM((half, n), jnp.float32),
            pltpu.VMEM((half, k), jnp.float32),
            pltpu.VMEM((half, k), jnp.float32),
            pltpu.SemaphoreType.DMA((2,)),
            pltpu.SemaphoreType.DMA((2,)),
            pltpu.SemaphoreType.DMA((8,)),
            pltpu.SemaphoreType.DMA((8,)),
            pltpu.SemaphoreType.DMA((8,)),
        ],
        compiler_params=pltpu.CompilerParams(
            collective_id=0, vmem_limit_bytes=100 * 1024 * 1024
        ),
    )(A, B)
